# baseline (speedup 1.0000x reference)
import functools

import jax
import jax.numpy as jnp
from jax import lax
from jax.experimental import pallas as pl
from jax.experimental.pallas import tpu as pltpu
from jax.experimental.pallas import tpu_sc as plsc

ROW_W = 128
K = 10
NC = 2
NS = 16
NW = NC * NS
TOK_BLK = 4096
Q = TOK_BLK // 4
UB = 8


def _tpack_body(t_ref, o_ref):
    y = t_ref[...].T
    for j in range(4):
        o_ref[:, 32 * j:32 * (j + 1)] = y[j * Q:(j + 1) * Q, :]


def _opack_body(x_ref, o_ref):
    x = x_ref[...]
    for k in range(UB):
        u = x[32 * k:32 * (k + 1), :]
        for j in range(4):
            c = 128 * k + 32 * j
            o_ref[0, :, c:c + 32] = u[:, 32 * j:32 * (j + 1)].T


def _emb_body(idx_hbm, table_hbm, out_hbm, idx_v, rows_v, sg0, sg1, so0, so1):
    wid = lax.axis_index("s") * NC + lax.axis_index("c")
    rows_total = idx_hbm.shape[0]
    per_w = rows_total // NW
    n_g = per_w // K
    base_row = wid * per_w

    sg = (sg0, sg1)
    so = (so0, so1)

    def fire(g, p):
        for j in range(K):
            pltpu.async_copy(
                table_hbm.at[idx_v.at[g * K + j]],
                rows_v.at[p].at[j],
                sg[p],
            )

    def drain_gathers(p):
        pltpu.make_async_copy(out_hbm.at[pl.ds(0, K)], rows_v.at[p], sg[p]).wait()

    def start_writeback(g, p):
        pltpu.async_copy(rows_v.at[p], out_hbm.at[pl.ds(base_row + g * K, K)], so[p])

    def wait_writeback(p):
        pltpu.make_async_copy(rows_v.at[p], out_hbm.at[pl.ds(0, K)], so[p]).wait()

    pltpu.sync_copy(idx_hbm.at[pl.ds(base_row, per_w)], idx_v)
    fire(0, 0)

    def body(gg, carry):
        for p in (0, 1):
            g = 2 * gg + p

            @pl.when(g >= 1)
            def _():
                wait_writeback(1 - p)

            @pl.when(g + 1 < n_g)
            def _():
                fire(g + 1, 1 - p)

            drain_gathers(p)
            start_writeback(g, p)
        return carry

    lax.fori_loop(0, n_g // 2, body, 0)
    wait_writeback((n_g - 1) % 2)


def kernel(token_index, table):
    b, h = token_index.shape
    v, d = table.shape
    n = b * h
    n_units = n // ROW_W
    n_bt = b // ROW_W
    n_tb = (v + TOK_BLK - 1) // TOK_BLK
    vp = n_tb * TOK_BLK

    tpack = pl.pallas_call(
        _tpack_body,
        grid=(n_tb,),
        in_specs=[pl.BlockSpec((d, TOK_BLK), lambda i: (0, i))],
        out_specs=pl.BlockSpec((Q, 4 * d), lambda i: (i, 0)),
        out_shape=jax.ShapeDtypeStruct((n_tb * Q, 4 * d), jnp.float32),
    )
    packed = tpack(table.T)
    tlin = packed.reshape(vp, d)

    idx = token_index.T.reshape(n_units, 4, 32).transpose(0, 2, 1)
    idx = idx.reshape(n_units, ROW_W)
    idx = (idx & ~(TOK_BLK - 1)) | ((idx & (Q - 1)) << 2) | ((idx >> 10) & 3)

    per_w = n_units // NW
    mesh = plsc.VectorSubcoreMesh(core_axis_name="c", subcore_axis_name="s")
    fn = functools.partial(
        pl.kernel,
        mesh=mesh,
        out_type=jax.ShapeDtypeStruct((n_units, ROW_W, d), jnp.float32),
        scratch_types=[
            pltpu.VMEM((per_w, ROW_W), jnp.int32),
            pltpu.VMEM((2, K, ROW_W, d), jnp.float32),
            pltpu.SemaphoreType.DMA,
            pltpu.SemaphoreType.DMA,
            pltpu.SemaphoreType.DMA,
            pltpu.SemaphoreType.DMA,
        ],
        compiler_params=pltpu.CompilerParams(use_tc_tiling_on_sc=False),
    )(_emb_body)
    sc_out = fn(idx, tlin)

    opack = pl.pallas_call(
        _opack_body,
        grid=(h, n_bt // UB),
        in_specs=[
            pl.BlockSpec(
                (UB * d, ROW_W),
                lambda i, j: (i * (n_bt // UB) + j, 0),
            )
        ],
        out_specs=pl.BlockSpec((1, d, UB * ROW_W), lambda i, j: (i, 0, j)),
        out_shape=jax.ShapeDtypeStruct((h, d, b), jnp.float32),
    )
    y = opack(sc_out.reshape(n_units * d, ROW_W))
    return y.transpose(2, 0, 1)


# speedup vs baseline: 2.6076x; 2.6076x over previous
import functools

import jax
import jax.numpy as jnp
from jax import lax
from jax.experimental import pallas as pl
from jax.experimental.pallas import tpu as pltpu
from jax.experimental.pallas import tpu_sc as plsc

ROW_W = 128
K = 10
NC = 2
NS = 16
NW = NC * NS
TOK_BLK = 4096
Q = TOK_BLK // 4
UB = 8


def _tpack_body(t_ref, o_ref):
    y = t_ref[...].T
    o_ref[...] = jnp.concatenate(
        [y[j * Q:(j + 1) * Q, :] for j in range(4)], axis=1
    )


def _opack_body(x_ref, o_ref):
    x = x_ref[...]
    parts = []
    for k in range(UB):
        u = x[32 * k:32 * (k + 1), :]
        parts.extend(u[:, 32 * j:32 * (j + 1)].T for j in range(4))
    o_ref[0] = jnp.concatenate(parts, axis=1)


def _emb_body(idx_hbm, table_hbm, out_hbm, idx_v, rows_v, sg0, sg1, so0, so1):
    wid = lax.axis_index("s") * NC + lax.axis_index("c")
    rows_total = idx_hbm.shape[0]
    per_w = rows_total // NW
    n_g = per_w // K
    base_row = wid * per_w

    sg = (sg0, sg1)
    so = (so0, so1)

    def fire(g, p):
        for j in range(K):
            pltpu.async_copy(
                table_hbm.at[idx_v.at[g * K + j]],
                rows_v.at[p].at[j],
                sg[p],
            )

    def drain_gathers(p):
        pltpu.make_async_copy(out_hbm.at[pl.ds(0, K)], rows_v.at[p], sg[p]).wait()

    def start_writeback(g, p):
        pltpu.async_copy(rows_v.at[p], out_hbm.at[pl.ds(base_row + g * K, K)], so[p])

    def wait_writeback(p):
        pltpu.make_async_copy(rows_v.at[p], out_hbm.at[pl.ds(0, K)], so[p]).wait()

    pltpu.sync_copy(idx_hbm.at[pl.ds(base_row, per_w)], idx_v)
    fire(0, 0)

    def body(gg, carry):
        for p in (0, 1):
            g = 2 * gg + p

            @pl.when(g >= 1)
            def _():
                wait_writeback(1 - p)

            @pl.when(g + 1 < n_g)
            def _():
                fire(g + 1, 1 - p)

            drain_gathers(p)
            start_writeback(g, p)
        return carry

    lax.fori_loop(0, n_g // 2, body, 0)
    wait_writeback((n_g - 1) % 2)


def kernel(token_index, table):
    b, h = token_index.shape
    v, d = table.shape
    n = b * h
    n_units = n // ROW_W
    n_bt = b // ROW_W
    n_tb = (v + TOK_BLK - 1) // TOK_BLK
    vp = n_tb * TOK_BLK

    tpack = pl.pallas_call(
        _tpack_body,
        grid=(n_tb,),
        in_specs=[pl.BlockSpec((d, TOK_BLK), lambda i: (0, i))],
        out_specs=pl.BlockSpec((Q, 4 * d), lambda i: (i, 0)),
        out_shape=jax.ShapeDtypeStruct((n_tb * Q, 4 * d), jnp.float32),
    )
    packed = tpack(table.T)
    tlin = packed.reshape(vp, d)

    idx = token_index.T.reshape(n_units, 4, 32).transpose(0, 2, 1)
    idx = idx.reshape(n_units, ROW_W)
    idx = (idx & ~(TOK_BLK - 1)) | ((idx & (Q - 1)) << 2) | ((idx >> 10) & 3)

    per_w = n_units // NW
    mesh = plsc.VectorSubcoreMesh(core_axis_name="c", subcore_axis_name="s")
    fn = functools.partial(
        pl.kernel,
        mesh=mesh,
        out_type=jax.ShapeDtypeStruct((n_units, ROW_W, d), jnp.float32),
        scratch_types=[
            pltpu.VMEM((per_w, ROW_W), jnp.int32),
            pltpu.VMEM((2, K, ROW_W, d), jnp.float32),
            pltpu.SemaphoreType.DMA,
            pltpu.SemaphoreType.DMA,
            pltpu.SemaphoreType.DMA,
            pltpu.SemaphoreType.DMA,
        ],
        compiler_params=pltpu.CompilerParams(use_tc_tiling_on_sc=False),
    )(_emb_body)
    sc_out = fn(idx, tlin)

    opack = pl.pallas_call(
        _opack_body,
        grid=(h, n_bt // UB),
        in_specs=[
            pl.BlockSpec(
                (UB * d, ROW_W),
                lambda i, j: (i * (n_bt // UB) + j, 0),
            )
        ],
        out_specs=pl.BlockSpec((1, d, UB * ROW_W), lambda i, j: (i, 0, j)),
        out_shape=jax.ShapeDtypeStruct((h, d, b), jnp.float32),
    )
    y = opack(sc_out.reshape(n_units * d, ROW_W))
    return y.transpose(2, 0, 1)


# speedup vs baseline: 3.9058x; 1.4979x over previous
import functools

import jax
import jax.numpy as jnp
from jax import lax
from jax.experimental import pallas as pl
from jax.experimental.pallas import tpu as pltpu
from jax.experimental.pallas import tpu_sc as plsc

ROW_W = 128
K = 10
NC = 2
NS = 16
NW = NC * NS
TOK_BLK = 8192
Q = TOK_BLK // 4
QS = Q.bit_length() - 1
UB = 32


def _tpack_body(t_ref, o_ref):
    y = t_ref[...].T
    o_ref[...] = jnp.concatenate(
        [y[j * Q:(j + 1) * Q, :] for j in range(4)], axis=1
    )


def _opack_body(x_ref, o_ref):
    x = x_ref[...]
    parts = []
    for k in range(UB):
        u = x[32 * k:32 * (k + 1), :]
        parts.extend(u[:, 32 * j:32 * (j + 1)].T for j in range(4))
    o_ref[0] = jnp.concatenate(parts, axis=1)


def _emb_body(idx_hbm, table_hbm, out_hbm, idx_v, rows_v, sg0, sg1, so0, so1):
    wid = lax.axis_index("s") * NC + lax.axis_index("c")
    rows_total = idx_hbm.shape[0]
    per_w = rows_total // NW
    n_g = per_w // K
    base_row = wid * per_w

    sg = (sg0, sg1)
    so = (so0, so1)

    def fire(g, p):
        for j in range(K):
            pltpu.async_copy(
                table_hbm.at[idx_v.at[g * K + j]],
                rows_v.at[p].at[j],
                sg[p],
            )

    def drain_gathers(p):
        pltpu.make_async_copy(out_hbm.at[pl.ds(0, K)], rows_v.at[p], sg[p]).wait()

    def start_writeback(g, p):
        pltpu.async_copy(rows_v.at[p], out_hbm.at[pl.ds(base_row + g * K, K)], so[p])

    def wait_writeback(p):
        pltpu.make_async_copy(rows_v.at[p], out_hbm.at[pl.ds(0, K)], so[p]).wait()

    pltpu.sync_copy(idx_hbm.at[pl.ds(base_row, per_w)], idx_v)
    fire(0, 0)

    def body(gg, carry):
        for p in (0, 1):
            g = 2 * gg + p

            @pl.when(g >= 1)
            def _():
                wait_writeback(1 - p)

            @pl.when(g + 1 < n_g)
            def _():
                fire(g + 1, 1 - p)

            drain_gathers(p)
            start_writeback(g, p)
        return carry

    lax.fori_loop(0, n_g // 2, body, 0)
    wait_writeback((n_g - 1) % 2)


def kernel(token_index, table):
    b, h = token_index.shape
    v, d = table.shape
    n = b * h
    n_units = n // ROW_W
    n_bt = b // ROW_W
    n_tb = (v + TOK_BLK - 1) // TOK_BLK
    vp = n_tb * TOK_BLK

    tpack = pl.pallas_call(
        _tpack_body,
        grid=(n_tb,),
        in_specs=[pl.BlockSpec((d, TOK_BLK), lambda i: (0, i))],
        out_specs=pl.BlockSpec((Q, 4 * d), lambda i: (i, 0)),
        out_shape=jax.ShapeDtypeStruct((n_tb * Q, 4 * d), jnp.float32),
    )
    packed = tpack(table.T)
    tlin = packed.reshape(vp, d)

    idx = token_index.T.reshape(n_units, 4, 32).transpose(0, 2, 1)
    idx = idx.reshape(n_units, ROW_W)
    idx = (idx & ~(TOK_BLK - 1)) | ((idx & (Q - 1)) << 2) | ((idx >> QS) & 3)

    per_w = n_units // NW
    mesh = plsc.VectorSubcoreMesh(core_axis_name="c", subcore_axis_name="s")
    fn = functools.partial(
        pl.kernel,
        mesh=mesh,
        out_type=jax.ShapeDtypeStruct((n_units, ROW_W, d), jnp.float32),
        scratch_types=[
            pltpu.VMEM((per_w, ROW_W), jnp.int32),
            pltpu.VMEM((2, K, ROW_W, d), jnp.float32),
            pltpu.SemaphoreType.DMA,
            pltpu.SemaphoreType.DMA,
            pltpu.SemaphoreType.DMA,
            pltpu.SemaphoreType.DMA,
        ],
        compiler_params=pltpu.CompilerParams(use_tc_tiling_on_sc=False),
    )(_emb_body)
    sc_out = fn(idx, tlin)

    opack = pl.pallas_call(
        _opack_body,
        grid=(h, n_bt // UB),
        in_specs=[
            pl.BlockSpec(
                (UB * d, ROW_W),
                lambda i, j: (i * (n_bt // UB) + j, 0),
            )
        ],
        out_specs=pl.BlockSpec((1, d, UB * ROW_W), lambda i, j: (i, 0, j)),
        out_shape=jax.ShapeDtypeStruct((h, d, b), jnp.float32),
    )
    y = opack(sc_out.reshape(n_units * d, ROW_W))
    return y.transpose(2, 0, 1)


# speedup vs baseline: 4.0392x; 1.0342x over previous
import functools

import jax
import jax.numpy as jnp
from jax import lax
from jax.experimental import pallas as pl
from jax.experimental.pallas import tpu as pltpu
from jax.experimental.pallas import tpu_sc as plsc

ROW_W = 128
K = 10
NC = 2
NS = 16
NW = NC * NS
TOK_BLK = 16384
Q = TOK_BLK // 4
QS = Q.bit_length() - 1
UB = 64


def _tpack_body(t_ref, o_ref):
    y = t_ref[...].T
    o_ref[...] = jnp.concatenate(
        [y[j * Q:(j + 1) * Q, :] for j in range(4)], axis=1
    )


def _opack_body(x_ref, o_ref):
    x = x_ref[...]
    parts = []
    for k in range(UB):
        u = x[32 * k:32 * (k + 1), :]
        parts.extend(u[:, 32 * j:32 * (j + 1)].T for j in range(4))
    o_ref[0] = jnp.concatenate(parts, axis=1)


def _emb_body(idx_hbm, table_hbm, out_hbm, idx_v, rows_v, sg0, sg1, so0, so1):
    wid = lax.axis_index("s") * NC + lax.axis_index("c")
    rows_total = idx_hbm.shape[0]
    per_w = rows_total // NW
    n_g = per_w // K
    base_row = wid * per_w

    sg = (sg0, sg1)
    so = (so0, so1)

    def fire(g, p):
        for j in range(K):
            pltpu.async_copy(
                table_hbm.at[idx_v.at[g * K + j]],
                rows_v.at[p].at[j],
                sg[p],
            )

    def drain_gathers(p):
        pltpu.make_async_copy(out_hbm.at[pl.ds(0, K)], rows_v.at[p], sg[p]).wait()

    def start_writeback(g, p):
        pltpu.async_copy(rows_v.at[p], out_hbm.at[pl.ds(base_row + g * K, K)], so[p])

    def wait_writeback(p):
        pltpu.make_async_copy(rows_v.at[p], out_hbm.at[pl.ds(0, K)], so[p]).wait()

    pltpu.sync_copy(idx_hbm.at[pl.ds(base_row, per_w)], idx_v)
    fire(0, 0)

    def body(gg, carry):
        for p in (0, 1):
            g = 2 * gg + p

            @pl.when(g >= 1)
            def _():
                wait_writeback(1 - p)

            @pl.when(g + 1 < n_g)
            def _():
                fire(g + 1, 1 - p)

            drain_gathers(p)
            start_writeback(g, p)
        return carry

    lax.fori_loop(0, n_g // 2, body, 0)
    wait_writeback((n_g - 1) % 2)


def kernel(token_index, table):
    b, h = token_index.shape
    v, d = table.shape
    n = b * h
    n_units = n // ROW_W
    n_bt = b // ROW_W
    n_tb = (v + TOK_BLK - 1) // TOK_BLK
    vp = n_tb * TOK_BLK

    tpack = pl.pallas_call(
        _tpack_body,
        grid=(n_tb,),
        in_specs=[pl.BlockSpec((d, TOK_BLK), lambda i: (0, i))],
        out_specs=pl.BlockSpec((Q, 4 * d), lambda i: (i, 0)),
        out_shape=jax.ShapeDtypeStruct((n_tb * Q, 4 * d), jnp.float32),
    )
    packed = tpack(table.T)
    tlin = packed.reshape(vp, d)

    idx = token_index.T.reshape(n_units, 4, 32).transpose(0, 2, 1)
    idx = idx.reshape(n_units, ROW_W)
    idx = (idx & ~(TOK_BLK - 1)) | ((idx & (Q - 1)) << 2) | ((idx >> QS) & 3)

    per_w = n_units // NW
    mesh = plsc.VectorSubcoreMesh(core_axis_name="c", subcore_axis_name="s")
    fn = functools.partial(
        pl.kernel,
        mesh=mesh,
        out_type=jax.ShapeDtypeStruct((n_units, ROW_W, d), jnp.float32),
        scratch_types=[
            pltpu.VMEM((per_w, ROW_W), jnp.int32),
            pltpu.VMEM((2, K, ROW_W, d), jnp.float32),
            pltpu.SemaphoreType.DMA,
            pltpu.SemaphoreType.DMA,
            pltpu.SemaphoreType.DMA,
            pltpu.SemaphoreType.DMA,
        ],
        compiler_params=pltpu.CompilerParams(use_tc_tiling_on_sc=False),
    )(_emb_body)
    sc_out = fn(idx, tlin)

    opack = pl.pallas_call(
        _opack_body,
        grid=(h, n_bt // UB),
        in_specs=[
            pl.BlockSpec(
                (UB * d, ROW_W),
                lambda i, j: (i * (n_bt // UB) + j, 0),
            )
        ],
        out_specs=pl.BlockSpec((1, d, UB * ROW_W), lambda i, j: (i, 0, j)),
        out_shape=jax.ShapeDtypeStruct((h, d, b), jnp.float32),
    )
    y = opack(sc_out.reshape(n_units * d, ROW_W))
    return y.transpose(2, 0, 1)


# speedup vs baseline: 4.1126x; 1.0182x over previous
import functools

import jax
import jax.numpy as jnp
from jax import lax
from jax.experimental import pallas as pl
from jax.experimental.pallas import tpu as pltpu
from jax.experimental.pallas import tpu_sc as plsc

ROW_W = 128
K = 10
NC = 2
NS = 16
NW = NC * NS
TOK_BLK = 32768
Q = TOK_BLK // 4
QS = Q.bit_length() - 1
UB = 128


def _tpack_body(t_ref, o_ref):
    y = t_ref[...].T
    o_ref[...] = jnp.concatenate(
        [y[j * Q:(j + 1) * Q, :] for j in range(4)], axis=1
    )


def _opack_body(x_ref, o_ref):
    x = x_ref[...]
    parts = []
    for k in range(UB):
        u = x[32 * k:32 * (k + 1), :]
        parts.extend(u[:, 32 * j:32 * (j + 1)].T for j in range(4))
    o_ref[0] = jnp.concatenate(parts, axis=1)


def _emb_body(idx_hbm, table_hbm, out_hbm, idx_v, rows_v, sg0, sg1, so0, so1):
    wid = lax.axis_index("s") * NC + lax.axis_index("c")
    rows_total = idx_hbm.shape[0]
    per_w = rows_total // NW
    n_g = per_w // K
    base_row = wid * per_w

    sg = (sg0, sg1)
    so = (so0, so1)

    def fire(g, p):
        for j in range(K):
            pltpu.async_copy(
                table_hbm.at[idx_v.at[g * K + j]],
                rows_v.at[p].at[j],
                sg[p],
            )

    def drain_gathers(p):
        pltpu.make_async_copy(out_hbm.at[pl.ds(0, K)], rows_v.at[p], sg[p]).wait()

    def start_writeback(g, p):
        pltpu.async_copy(rows_v.at[p], out_hbm.at[pl.ds(base_row + g * K, K)], so[p])

    def wait_writeback(p):
        pltpu.make_async_copy(rows_v.at[p], out_hbm.at[pl.ds(0, K)], so[p]).wait()

    pltpu.sync_copy(idx_hbm.at[pl.ds(base_row, per_w)], idx_v)
    fire(0, 0)

    def body(gg, carry):
        for p in (0, 1):
            g = 2 * gg + p

            @pl.when(g >= 1)
            def _():
                wait_writeback(1 - p)

            @pl.when(g + 1 < n_g)
            def _():
                fire(g + 1, 1 - p)

            drain_gathers(p)
            start_writeback(g, p)
        return carry

    lax.fori_loop(0, n_g // 2, body, 0)
    wait_writeback((n_g - 1) % 2)


def kernel(token_index, table):
    b, h = token_index.shape
    v, d = table.shape
    n = b * h
    n_units = n // ROW_W
    n_bt = b // ROW_W
    n_tb = (v + TOK_BLK - 1) // TOK_BLK
    vp = n_tb * TOK_BLK

    tpack = pl.pallas_call(
        _tpack_body,
        grid=(n_tb,),
        in_specs=[pl.BlockSpec((d, TOK_BLK), lambda i: (0, i))],
        out_specs=pl.BlockSpec((Q, 4 * d), lambda i: (i, 0)),
        out_shape=jax.ShapeDtypeStruct((n_tb * Q, 4 * d), jnp.float32),
    )
    packed = tpack(table.T)
    tlin = packed.reshape(vp, d)

    idx = token_index.T.reshape(n_units, 4, 32).transpose(0, 2, 1)
    idx = idx.reshape(n_units, ROW_W)
    idx = (idx & ~(TOK_BLK - 1)) | ((idx & (Q - 1)) << 2) | ((idx >> QS) & 3)

    per_w = n_units // NW
    mesh = plsc.VectorSubcoreMesh(core_axis_name="c", subcore_axis_name="s")
    fn = functools.partial(
        pl.kernel,
        mesh=mesh,
        out_type=jax.ShapeDtypeStruct((n_units, ROW_W, d), jnp.float32),
        scratch_types=[
            pltpu.VMEM((per_w, ROW_W), jnp.int32),
            pltpu.VMEM((2, K, ROW_W, d), jnp.float32),
            pltpu.SemaphoreType.DMA,
            pltpu.SemaphoreType.DMA,
            pltpu.SemaphoreType.DMA,
            pltpu.SemaphoreType.DMA,
        ],
        compiler_params=pltpu.CompilerParams(use_tc_tiling_on_sc=False),
    )(_emb_body)
    sc_out = fn(idx, tlin)

    opack = pl.pallas_call(
        _opack_body,
        grid=(h, n_bt // UB),
        in_specs=[
            pl.BlockSpec(
                (UB * d, ROW_W),
                lambda i, j: (i * (n_bt // UB) + j, 0),
            )
        ],
        out_specs=pl.BlockSpec((1, d, UB * ROW_W), lambda i, j: (i, 0, j)),
        out_shape=jax.ShapeDtypeStruct((h, d, b), jnp.float32),
    )
    y = opack(sc_out.reshape(n_units * d, ROW_W))
    return y.transpose(2, 0, 1)
